# Initial kernel scaffold; baseline (speedup 1.0000x reference)
import jax
import jax.numpy as jnp
from jax.experimental import pallas as pl


def kernel(x, train_user, train_item):
    raise NotImplementedError("write your pallas kernel here")


# speedup vs baseline: 6.1050x; 6.1050x over previous
import functools

import jax
import jax.numpy as jnp
from jax import lax
from jax.experimental import pallas as pl
from jax.experimental.pallas import tpu as pltpu
from jax.experimental.pallas import tpu_sc as plsc

NU = 50000
NI = 50000
NN = NU + NI
D = 128
E0 = 312500

NTILES = 16
BLK = 512
NBLK = 39
TILE_E = NBLK * BLK
E0P = NTILES * TILE_E

CHUNK = 7936
NPASS = 4
STRIPE = CHUNK // NTILES
HPAD = NTILES * 3200

_MESH = dict(core_axis_name="c", subcore_axis_name="s", num_cores=2,
             num_subcores=16)


def _deg_body(da_h, db_h, ones_h, z_h, deg_h, ibuf, ones_v, hist):
    c = lax.axis_index("c")
    s = lax.axis_index("s")
    pltpu.sync_copy(ones_h, ones_v)
    pltpu.sync_copy(z_h, hist.at[pl.ds(s * 3200, 3200)])
    plsc.subcore_barrier()

    def count_edges(arr_h):
        def block(b, carry):
            off = pl.multiple_of(s * TILE_E + b * BLK, 128)
            for j in range(4):
                pltpu.sync_copy(arr_h.at[pl.ds(off + j * 128, 128)],
                                ibuf.at[j])
            for j in range(4):
                pltpu.sync_copy(
                    ones_v,
                    hist.at[plsc.Indices(ibuf.at[j], ignored_value=-1)],
                    add=True)
            return carry
        lax.fori_loop(0, NBLK, block, 0)

    pl.when(c == 0)(lambda: count_edges(db_h))
    pl.when(c == 1)(lambda: count_edges(da_h))
    plsc.subcore_barrier()
    pltpu.sync_copy(hist.at[pl.ds(s * 3200, 3200)],
                    deg_h.at[pl.ds(c * HPAD + s * 3200, 3200)])


def _deg(da, db, ones128, z3200):
    return pl.kernel(
        _deg_body,
        out_type=jax.ShapeDtypeStruct((2 * HPAD,), jnp.float32),
        mesh=plsc.VectorSubcoreMesh(**_MESH),
        scratch_types=[
            pltpu.VMEM((4, 128), jnp.int32),
            pltpu.VMEM((128,), jnp.float32),
            pltpu.VMEM_SHARED((HPAD,), jnp.float32),
        ],
    )(da, db, ones128, z3200)


def _spmm_body(y_h, sa_h, da_h, sb_h, db_h, z_h, out_h,
               sbuf, dbuf, rows, acc, gsem, ssem):
    c = lax.axis_index("c")
    s = lax.axis_index("s")
    wb_prev = [None]
    for half in range(2):
        src_h = (sa_h, sb_h)[half]
        dst_h = (da_h, db_h)[half]
        out_off = NU if half == 0 else 0
        for p in range(NPASS):
            idx3 = 2 * p + c
            base = jnp.minimum(idx3 * CHUNK, NU - CHUNK)
            if wb_prev[0] is not None:
                wb_prev[0].wait()
            pltpu.sync_copy(z_h, acc.at[pl.ds(s * STRIPE, STRIPE)])
            plsc.subcore_barrier()

            def block(b, carry):
                off = pl.multiple_of(s * TILE_E + b * BLK, 128)
                for j in range(4):
                    pltpu.sync_copy(src_h.at[pl.ds(off + j * 128, 128)],
                                    sbuf.at[j])
                    pltpu.sync_copy(dst_h.at[pl.ds(off + j * 128, 128)],
                                    dbuf.at[j])
                for j in range(4):
                    for k in range(8):
                        dv = dbuf[j, pl.ds(k * 16, 16)]
                        sv = sbuf[j, pl.ds(k * 16, 16)]
                        m = (dv >= base) & (dv < base + CHUNK)
                        dbuf[j, pl.ds(k * 16, 16)] = jnp.where(m, dv - base,
                                                               -1)
                        sbuf[j, pl.ds(k * 16, 16)] = jnp.where(m, sv, -1)
                gd = []
                for j in range(4):
                    gd.append(pltpu.async_copy(
                        y_h.at[plsc.Indices(sbuf.at[j], ignored_value=-1)],
                        rows.at[pl.ds(j * 128, 128)], gsem))
                sd = []
                for j in range(4):
                    gd[j].wait()
                    sd.append(pltpu.async_copy(
                        rows.at[pl.ds(j * 128, 128)],
                        acc.at[plsc.Indices(dbuf.at[j], ignored_value=-1)],
                        ssem, add=True))
                for dsc in sd:
                    dsc.wait()
                return carry

            lax.fori_loop(0, NBLK, block, 0)
            plsc.subcore_barrier()
            wb_prev[0] = pltpu.async_copy(
                acc.at[pl.ds(s * STRIPE, STRIPE)],
                out_h.at[pl.ds(out_off + base + s * STRIPE, STRIPE)],
                gsem)
    wb_prev[0].wait()


def _spmm(y, sa, da, sb, db, z800):
    return pl.kernel(
        _spmm_body,
        out_type=jax.ShapeDtypeStruct((NN, D), jnp.float32),
        mesh=plsc.VectorSubcoreMesh(**_MESH),
        scratch_types=[
            pltpu.VMEM((4, 128), jnp.int32),
            pltpu.VMEM((4, 128), jnp.int32),
            pltpu.VMEM((BLK, D), jnp.float32),
            pltpu.VMEM_SHARED((CHUNK, D), jnp.float32),
            pltpu.SemaphoreType.DMA,
            pltpu.SemaphoreType.DMA,
        ],
    )(y, sa, da, sb, db, z800)


def _scale_body(d_ref, v_ref, o_ref):
    d = d_ref[...]
    w = lax.rsqrt(jnp.where(d == 0.0, 1e-6, d))
    o_ref[...] = v_ref[...] * w


def _scale(v, deg2d):
    return pl.pallas_call(
        _scale_body,
        grid=(NN // 400,),
        in_specs=[
            pl.BlockSpec((400, 1), lambda i: (i, 0)),
            pl.BlockSpec((400, D), lambda i: (i, 0)),
        ],
        out_specs=pl.BlockSpec((400, D), lambda i: (i, 0)),
        out_shape=jax.ShapeDtypeStruct((NN, D), jnp.float32),
    )(deg2d, v)


def kernel(x, train_user, train_item):
    pad = jnp.full((E0P - E0,), -1, jnp.int32)
    sa = jnp.concatenate([train_user, pad])
    da = jnp.concatenate([train_item, pad])
    sb = jnp.concatenate([train_item + NU, pad])
    db = jnp.concatenate([train_user, pad])
    ones128 = jnp.ones((128,), jnp.float32)
    z3200 = jnp.zeros((3200,), jnp.float32)
    z800 = jnp.zeros((STRIPE, D), jnp.float32)

    degp = _deg(da, db, ones128, z3200)
    deg2d = jnp.concatenate([degp[:NU], degp[HPAD:HPAD + NI]])[:, None]
    y = _scale(x, deg2d)
    acc = _spmm(y, sa, da, sb, db, z800)
    return _scale(acc, deg2d)


# speedup vs baseline: 11.4867x; 1.8815x over previous
import functools

import jax
import jax.numpy as jnp
from jax import lax
from jax.experimental import pallas as pl
from jax.experimental.pallas import tpu as pltpu
from jax.experimental.pallas import tpu_sc as plsc

NU = 50000
NI = 50000
NN = NU + NI
D = 128
E0 = 312500

NTILES = 16
BLK = 256
NBLK = 78
NSLOT = 2
TILE_E = NBLK * BLK
E0P = NTILES * TILE_E

CHUNK = 7808
NPASS = 4
STRIPE = CHUNK // NTILES
HPAD = NTILES * 3200

_MESH = dict(core_axis_name="c", subcore_axis_name="s", num_cores=2,
             num_subcores=16)


def _deg_body(da_h, db_h, ones_h, z_h, deg_h, ibuf, ones_v, hist):
    c = lax.axis_index("c")
    s = lax.axis_index("s")
    pltpu.sync_copy(ones_h, ones_v)
    pltpu.sync_copy(z_h, hist.at[pl.ds(s * 3200, 3200)])
    plsc.subcore_barrier()

    def count_edges(arr_h):
        def block(b, carry):
            off = pl.multiple_of(s * TILE_E + b * BLK, 128)
            for j in range(2):
                pltpu.sync_copy(arr_h.at[pl.ds(off + j * 128, 128)],
                                ibuf.at[j])
            for j in range(2):
                pltpu.sync_copy(
                    ones_v,
                    hist.at[plsc.Indices(ibuf.at[j], ignored_value=-1)],
                    add=True)
            return carry
        lax.fori_loop(0, NBLK, block, 0)

    pl.when(c == 0)(lambda: count_edges(db_h))
    pl.when(c == 1)(lambda: count_edges(da_h))
    plsc.subcore_barrier()
    pltpu.sync_copy(hist.at[pl.ds(s * 3200, 3200)],
                    deg_h.at[pl.ds(c * HPAD + s * 3200, 3200)])


def _deg(da, db, ones128, z3200):
    return pl.kernel(
        _deg_body,
        out_type=jax.ShapeDtypeStruct((2 * HPAD,), jnp.float32),
        mesh=plsc.VectorSubcoreMesh(**_MESH),
        scratch_types=[
            pltpu.VMEM((2, 128), jnp.int32),
            pltpu.VMEM((128,), jnp.float32),
            pltpu.VMEM_SHARED((HPAD,), jnp.float32),
        ],
    )(da, db, ones128, z3200)


GROUPS = NBLK // NSLOT


def _spmm_body(y_h, sa_h, da_h, sb_h, db_h, z_h, out_h,
               rs0, rs1, rd0, rd1,
               fs0, fs1, fd0, fd1, rows, acc,
               st0, st1, g0, g1, s0, s1, wsem):
    c = lax.axis_index("c")
    s = lax.axis_index("s")
    raw_s = (rs0, rs1)
    raw_d = (rd0, rd1)
    fix_s = (fs0, fs1)
    fix_d = (fd0, fd1)
    stsems = (st0, st1)
    gsems = (g0, g1)
    ssems = (s0, s1)
    tile_off = s * TILE_E
    wb_prev = [None]
    for half in range(2):
        src_h = (sa_h, sb_h)[half]
        dst_h = (da_h, db_h)[half]
        out_off = NU if half == 0 else 0
        for p in range(NPASS):
            idx3 = 2 * p + c
            base = jnp.minimum(idx3 * CHUNK, NU - CHUNK)

            def stage(b, i):
                off = pl.multiple_of(tile_off + b * BLK, 128)
                pltpu.async_copy(src_h.at[pl.ds(off, BLK)], raw_s[i],
                                 stsems[i])
                pltpu.async_copy(dst_h.at[pl.ds(off, BLK)], raw_d[i],
                                 stsems[i])

            def wait_stage(i):
                pltpu.make_async_copy(src_h.at[pl.ds(0, BLK)], raw_s[i],
                                      stsems[i]).wait()
                pltpu.make_async_copy(dst_h.at[pl.ds(0, BLK)], raw_d[i],
                                      stsems[i]).wait()

            def fixup(i):
                for k in range(BLK // 16):
                    j, kk = k // 8, (k % 8) * 16
                    dv = raw_d[i][pl.ds(k * 16, 16)]
                    sv = raw_s[i][pl.ds(k * 16, 16)]
                    m = (dv >= base) & (dv < base + CHUNK)
                    fix_d[i][j, pl.ds(kk, 16)] = jnp.where(m, dv - base, -1)
                    fix_s[i][j, pl.ds(kk, 16)] = jnp.where(m, sv, -1)

            def gathers(i):
                for j in range(BLK // 128):
                    pltpu.async_copy(
                        y_h.at[plsc.Indices(fix_s[i].at[j],
                                            ignored_value=-1)],
                        rows.at[i, pl.ds(j * 128, 128)], gsems[i])

            def wait_gathers(i):
                for j in range(BLK // 128):
                    pltpu.make_async_copy(
                        y_h.at[plsc.Indices(fix_s[i].at[j],
                                            ignored_value=-1)],
                        rows.at[i, pl.ds(j * 128, 128)], gsems[i]).wait()

            def scatters(i):
                for j in range(BLK // 128):
                    pltpu.async_copy(
                        rows.at[i, pl.ds(j * 128, 128)],
                        acc.at[plsc.Indices(fix_d[i].at[j],
                                            ignored_value=-1)],
                        ssems[i], add=True)

            def wait_scatters(i):
                for j in range(BLK // 128):
                    pltpu.make_async_copy(
                        rows.at[i, pl.ds(j * 128, 128)],
                        acc.at[plsc.Indices(fix_d[i].at[j],
                                            ignored_value=-1)],
                        ssems[i]).wait()

            def drain_prev(i):
                def go():
                    wait_gathers(i)
                    scatters(i)
                return go

            if wb_prev[0] is not None:
                wb_prev[0].wait()
            pltpu.sync_copy(z_h, acc.at[pl.ds(s * STRIPE, STRIPE)])
            plsc.subcore_barrier()

            for i in range(NSLOT):
                stage(i, i)

            def group(g, carry):
                for i in range(NSLOT):
                    b = g * NSLOT + i
                    pl.when(g > 0)(lambda i=i: wait_scatters(i))
                    wait_stage(i)
                    fixup(i)
                    gathers(i)
                    pl.when(g < GROUPS - 1)(lambda b=b, i=i: stage(b + NSLOT,
                                                                   i))
                    if i == 0:
                        pl.when(g > 0)(drain_prev(NSLOT - 1))
                    else:
                        drain_prev(i - 1)()
                return carry

            lax.fori_loop(0, GROUPS, group, 0)
            drain_prev(NSLOT - 1)()
            for i in range(NSLOT):
                wait_scatters(i)
            plsc.subcore_barrier()
            wb_prev[0] = pltpu.async_copy(
                acc.at[pl.ds(s * STRIPE, STRIPE)],
                out_h.at[pl.ds(out_off + base + s * STRIPE, STRIPE)],
                wsem)
    wb_prev[0].wait()


def _spmm(y, sa, da, sb, db, z800):
    return pl.kernel(
        _spmm_body,
        out_type=jax.ShapeDtypeStruct((NN, D), jnp.float32),
        mesh=plsc.VectorSubcoreMesh(**_MESH),
        scratch_types=(
            [pltpu.VMEM((BLK,), jnp.int32)] * 4
            + [pltpu.VMEM((BLK // 128, 128), jnp.int32)] * 4
            + [
                pltpu.VMEM((NSLOT, BLK, D), jnp.float32),
                pltpu.VMEM_SHARED((CHUNK, D), jnp.float32),
            ]
            + [pltpu.SemaphoreType.DMA] * 7
        ),
    )(y, sa, da, sb, db, z800)


def _scale_body(d_ref, v_ref, o_ref):
    d = d_ref[...]
    w = lax.rsqrt(jnp.where(d == 0.0, 1e-6, d))
    o_ref[...] = v_ref[...] * w


def _scale(v, deg2d):
    return pl.pallas_call(
        _scale_body,
        grid=(NN // 400,),
        in_specs=[
            pl.BlockSpec((400, 1), lambda i: (i, 0)),
            pl.BlockSpec((400, D), lambda i: (i, 0)),
        ],
        out_specs=pl.BlockSpec((400, D), lambda i: (i, 0)),
        out_shape=jax.ShapeDtypeStruct((NN, D), jnp.float32),
    )(deg2d, v)


def kernel(x, train_user, train_item):
    pad = jnp.full((E0P - E0,), -1, jnp.int32)
    sa = jnp.concatenate([train_user, pad])
    da = jnp.concatenate([train_item, pad])
    sb = jnp.concatenate([train_item + NU, pad])
    db = jnp.concatenate([train_user, pad])
    ones128 = jnp.ones((128,), jnp.float32)
    z3200 = jnp.zeros((3200,), jnp.float32)
    z800 = jnp.zeros((STRIPE, D), jnp.float32)

    degp = _deg(da, db, ones128, z3200)
    deg2d = jnp.concatenate([degp[:NU], degp[HPAD:HPAD + NI]])[:, None]
    y = _scale(x, deg2d)
    acc = _spmm(y, sa, da, sb, db, z800)
    return _scale(acc, deg2d)


# speedup vs baseline: 12.0752x; 1.0512x over previous
import functools

import jax
import jax.numpy as jnp
from jax import lax
from jax.experimental import pallas as pl
from jax.experimental.pallas import tpu as pltpu
from jax.experimental.pallas import tpu_sc as plsc

NU = 50000
NI = 50000
NN = NU + NI
D = 128
E0 = 312500

NTILES = 16
BLK = 128
NBLK = 153
NSLOT = 3
TILE_E = NBLK * BLK
E0P = NTILES * TILE_E

CHUNK = 7936
NPASS = 4
STRIPE = CHUNK // NTILES
HPAD = NTILES * 3200

_MESH = dict(core_axis_name="c", subcore_axis_name="s", num_cores=2,
             num_subcores=16)


def _deg_body(da_h, db_h, ones_h, z_h, deg_h, ibuf, ones_v, hist):
    c = lax.axis_index("c")
    s = lax.axis_index("s")
    pltpu.sync_copy(ones_h, ones_v)
    pltpu.sync_copy(z_h, hist.at[pl.ds(s * 3200, 3200)])
    plsc.subcore_barrier()

    def count_edges(arr_h):
        def block(b, carry):
            off = pl.multiple_of(s * TILE_E + b * BLK, 128)
            for j in range(BLK // 128):
                pltpu.sync_copy(arr_h.at[pl.ds(off + j * 128, 128)],
                                ibuf.at[j])
            for j in range(BLK // 128):
                pltpu.sync_copy(
                    ones_v,
                    hist.at[plsc.Indices(ibuf.at[j], ignored_value=-1)],
                    add=True)
            return carry
        lax.fori_loop(0, NBLK, block, 0)

    pl.when(c == 0)(lambda: count_edges(db_h))
    pl.when(c == 1)(lambda: count_edges(da_h))
    plsc.subcore_barrier()
    pltpu.sync_copy(hist.at[pl.ds(s * 3200, 3200)],
                    deg_h.at[pl.ds(c * HPAD + s * 3200, 3200)])


def _deg(da, db, ones128, z3200):
    return pl.kernel(
        _deg_body,
        out_type=jax.ShapeDtypeStruct((2 * HPAD,), jnp.float32),
        mesh=plsc.VectorSubcoreMesh(**_MESH),
        scratch_types=[
            pltpu.VMEM((2, 128), jnp.int32),
            pltpu.VMEM((128,), jnp.float32),
            pltpu.VMEM_SHARED((HPAD,), jnp.float32),
        ],
    )(da, db, ones128, z3200)


GROUPS = NBLK // NSLOT


def _spmm_body(y_h, sa_h, da_h, sb_h, db_h, z_h, out_h,
               rs0, rs1, rs2, rd0, rd1, rd2,
               gs0, gs1, gs2, gd0, gd1, gd2, rows, acc,
               st0, st1, st2, g0, g1, g2, s0, s1, s2, wsem):
    c = lax.axis_index("c")
    s = lax.axis_index("s")
    raw_s = (rs0, rs1, rs2)
    raw_d = (rd0, rd1, rd2)
    gb_s = (gs0, gs1, gs2)
    gb_d = (gd0, gd1, gd2)
    stsems = (st0, st1, st2)
    gsems = (g0, g1, g2)
    ssems = (s0, s1, s2)
    tile_off = s * TILE_E
    wb_prev = [None]
    for half in range(2):
        src_h = (sa_h, sb_h)[half]
        dst_h = (da_h, db_h)[half]
        out_off = NU if half == 0 else 0
        for p in range(NPASS):
            idx3 = 2 * p + c
            base = jnp.minimum(idx3 * CHUNK, NU - CHUNK)

            def stage(b, i):
                off = pl.multiple_of(tile_off + b * BLK, 128)
                pltpu.async_copy(src_h.at[pl.ds(off, BLK)], raw_s[i],
                                 stsems[i])
                pltpu.async_copy(dst_h.at[pl.ds(off, BLK)], raw_d[i],
                                 stsems[i])

            def wait_stage(i):
                pltpu.make_async_copy(src_h.at[pl.ds(0, BLK)], raw_s[i],
                                      stsems[i]).wait()
                pltpu.make_async_copy(dst_h.at[pl.ds(0, BLK)], raw_d[i],
                                      stsems[i]).wait()

            def transform(i):
                for k in range(BLK // 16):
                    dv = raw_d[i][pl.ds(k * 16, 16)]
                    sv = raw_s[i][pl.ds(k * 16, 16)]
                    m = (dv >= base) & (dv < base + CHUNK)
                    gb_d[i][pl.ds(k * 16, 16)] = jnp.where(m, dv - base, -1)
                    gb_s[i][pl.ds(k * 16, 16)] = jnp.where(m, sv, -1)

            def gathers(i):
                pltpu.async_copy(
                    y_h.at[plsc.Indices(gb_s[i], ignored_value=-1)],
                    rows.at[i], gsems[i])

            def wait_gathers(i):
                pltpu.make_async_copy(
                    y_h.at[plsc.Indices(gb_s[i], ignored_value=-1)],
                    rows.at[i], gsems[i]).wait()

            def scatters(i):
                pltpu.async_copy(
                    rows.at[i],
                    acc.at[plsc.Indices(gb_d[i], ignored_value=-1)],
                    ssems[i], add=True)

            def wait_scatters(i):
                pltpu.make_async_copy(
                    rows.at[i],
                    acc.at[plsc.Indices(gb_d[i], ignored_value=-1)],
                    ssems[i]).wait()

            def drain_prev(i):
                def go():
                    wait_gathers(i)
                    scatters(i)
                return go

            if wb_prev[0] is not None:
                wb_prev[0].wait()
            pltpu.sync_copy(z_h, acc.at[pl.ds(s * STRIPE, STRIPE)])
            plsc.subcore_barrier()

            for i in range(NSLOT):
                stage(i, i)

            def group(g, carry):
                for i in range(NSLOT):
                    b = g * NSLOT + i
                    pl.when(g > 0)(lambda i=i: wait_scatters(i))
                    wait_stage(i)
                    transform(i)
                    gathers(i)
                    pl.when(g < GROUPS - 1)(lambda b=b, i=i: stage(b + NSLOT,
                                                                   i))
                    if i == 0:
                        pl.when(g > 0)(drain_prev(NSLOT - 1))
                    else:
                        drain_prev(i - 1)()
                return carry

            lax.fori_loop(0, GROUPS, group, 0)
            drain_prev(NSLOT - 1)()
            for i in range(NSLOT):
                wait_scatters(i)
            plsc.subcore_barrier()
            wb_prev[0] = pltpu.async_copy(
                acc.at[pl.ds(s * STRIPE, STRIPE)],
                out_h.at[pl.ds(out_off + base + s * STRIPE, STRIPE)],
                wsem)
    wb_prev[0].wait()


def _spmm(y, sa, da, sb, db, z800):
    return pl.kernel(
        _spmm_body,
        out_type=jax.ShapeDtypeStruct((NN, D), jnp.float32),
        mesh=plsc.VectorSubcoreMesh(**_MESH),
        scratch_types=(
            [pltpu.VMEM((BLK,), jnp.int32)] * 6
            + [pltpu.VMEM((128,), jnp.int32)] * 6
            + [
                pltpu.VMEM((NSLOT, 128, D), jnp.float32),
                pltpu.VMEM_SHARED((CHUNK, D), jnp.float32),
            ]
            + [pltpu.SemaphoreType.DMA] * 10
        ),
    )(y, sa, da, sb, db, z800)


def _scale_body(d_ref, v_ref, o_ref):
    d = d_ref[...]
    w = lax.rsqrt(jnp.where(d == 0.0, 1e-6, d))
    o_ref[...] = v_ref[...] * w


def _scale(v, deg2d):
    return pl.pallas_call(
        _scale_body,
        grid=(NN // 400,),
        in_specs=[
            pl.BlockSpec((400, 1), lambda i: (i, 0)),
            pl.BlockSpec((400, D), lambda i: (i, 0)),
        ],
        out_specs=pl.BlockSpec((400, D), lambda i: (i, 0)),
        out_shape=jax.ShapeDtypeStruct((NN, D), jnp.float32),
    )(deg2d, v)


def kernel(x, train_user, train_item):
    pad = jnp.full((E0P - E0,), -1, jnp.int32)
    sa = jnp.concatenate([train_user, pad])
    da = jnp.concatenate([train_item, pad])
    sb = jnp.concatenate([train_item + NU, pad])
    db = jnp.concatenate([train_user, pad])
    ones128 = jnp.ones((128,), jnp.float32)
    z3200 = jnp.zeros((3200,), jnp.float32)
    z800 = jnp.zeros((STRIPE, D), jnp.float32)

    degp = _deg(da, db, ones128, z3200)
    deg2d = jnp.concatenate([degp[:NU], degp[HPAD:HPAD + NI]])[:, None]
    y = _scale(x, deg2d)
    acc = _spmm(y, sa, da, sb, db, z800)
    return _scale(acc, deg2d)
